# Initial kernel scaffold; baseline (speedup 1.0000x reference)
import jax
import jax.numpy as jnp
from jax.experimental import pallas as pl


def kernel(x, Wq, bq, Wkv, bkv, Wo, bo):
    raise NotImplementedError("write your pallas kernel here")


# speedup vs baseline: 4.4872x; 4.4872x over previous
import jax
import jax.numpy as jnp
from jax.experimental import pallas as pl

BATCH = 16
DIM = 768
NH = 12
HD = 64
NLOC = 1024
NREG = 64
RS = 16
TOPK = 32
OUT_DIM = 2 * DIM
SCALE = DIM ** -0.5


def _bra_kernel(xs_ref, x0_ref, wqt_ref, bq_ref, wk_ref, wvt_ref, bv_ref,
                wot_ref, bo_ref, out_ref):
    f32 = jnp.float32
    xs = xs_ref[0]
    x0 = x0_ref[0]
    q = jnp.dot(x0, wqt_ref[...], preferred_element_type=f32) + bq_ref[...]

    k = jnp.dot(wk_ref[...], xs, preferred_element_type=f32)

    kr = k[:, 0:NREG]
    for j in range(1, RS):
        kr = jnp.maximum(kr, k[:, j * NREG:(j + 1) * NREG])
    a = jnp.dot(q, kr, preferred_element_type=f32)

    arow = jnp.broadcast_to(a, (NREG, NREG))
    acol = arow.T
    ii = jax.lax.broadcasted_iota(jnp.int32, (NREG, NREG), 0)
    jj = jax.lax.broadcasted_iota(jnp.int32, (NREG, NREG), 1)
    beats = (acol > arow) | ((acol == arow) & (ii < jj))
    rank = jnp.sum(beats.astype(f32), axis=0, keepdims=True)
    rankloc = jnp.concatenate([rank] * RS, axis=1)
    maskloc = rankloc < float(TOPK)

    hidx = jax.lax.broadcasted_iota(jnp.int32, (NH, DIM), 0)
    cidx = jax.lax.broadcasted_iota(jnp.int32, (NH, DIM), 1)
    diag = (cidx // HD) == hidx
    qm = jnp.where(diag, jnp.broadcast_to(q, (NH, DIM)), 0.0)
    s = jnp.dot(qm, k, preferred_element_type=f32) * SCALE
    s = jnp.where(maskloc, s, -1e30)
    s = s - jnp.max(s, axis=1, keepdims=True)
    e = jnp.exp(s)
    attn = e / jnp.sum(e, axis=1, keepdims=True)

    z = jax.lax.dot_general(attn, xs, (((1,), (1,)), ((), ())),
                            preferred_element_type=f32)

    o_full = jnp.dot(z, wvt_ref[...], preferred_element_type=f32)
    o_vec = jnp.sum(jnp.where(diag, o_full, 0.0), axis=0,
                    keepdims=True) + bv_ref[...]

    out_ref[0] = (jnp.dot(o_vec, wot_ref[...], preferred_element_type=f32)
                  + bo_ref[...])


def kernel(x, Wq, bq, Wkv, bkv, Wo, bo):
    xs = x[:, 1:].reshape(BATCH, DIM, 8, 4, 8, 4)
    xs = xs.transpose(0, 1, 3, 5, 2, 4).reshape(BATCH, DIM, NLOC)
    x0 = x[:, 0].reshape(BATCH, 1, DIM)
    wqt = Wq.T
    wk = Wkv[:DIM]
    wvt = Wkv[DIM:].T
    wot = Wo.T
    bq2 = bq.reshape(1, DIM)
    bv2 = bkv[DIM:].reshape(1, DIM)
    bo2 = bo.reshape(1, OUT_DIM)

    out = pl.pallas_call(
        _bra_kernel,
        grid=(BATCH,),
        in_specs=[
            pl.BlockSpec((1, DIM, NLOC), lambda b: (b, 0, 0)),
            pl.BlockSpec((1, 1, DIM), lambda b: (b, 0, 0)),
            pl.BlockSpec((DIM, DIM), lambda b: (0, 0)),
            pl.BlockSpec((1, DIM), lambda b: (0, 0)),
            pl.BlockSpec((DIM, DIM), lambda b: (0, 0)),
            pl.BlockSpec((DIM, DIM), lambda b: (0, 0)),
            pl.BlockSpec((1, DIM), lambda b: (0, 0)),
            pl.BlockSpec((DIM, OUT_DIM), lambda b: (0, 0)),
            pl.BlockSpec((1, OUT_DIM), lambda b: (0, 0)),
        ],
        out_specs=pl.BlockSpec((1, 1, OUT_DIM), lambda b: (b, 0, 0)),
        out_shape=jax.ShapeDtypeStruct((BATCH, 1, OUT_DIM), jnp.float32),
    )(xs, x0, wqt, bq2, wk, wvt, bv2, wot, bo2)
    return out.reshape(BATCH, OUT_DIM)


# speedup vs baseline: 5.2140x; 1.1620x over previous
import jax
import jax.numpy as jnp
from jax.experimental import pallas as pl

BATCH = 16
DIM = 768
NH = 12
HD = 64
NLOC = 1024
NREG = 64
RS = 16
TOPK = 32
OUT_DIM = 2 * DIM
SCALE = DIM ** -0.5

_NT = (((1,), (1,)), ((), ()))


def _rot(t, s):
    return jnp.concatenate([t[:, s:], t[:, :s]], axis=1)


def _bra_kernel(xs_ref, x0_ref, wq_ref, bq_ref, wkv_ref, bv_ref, wo_ref,
                bo_ref, out_ref):
    f32 = jnp.float32
    xs = xs_ref[0]
    x0 = x0_ref[0]
    q = jax.lax.dot_general(x0, wq_ref[...], _NT,
                            preferred_element_type=f32) + bq_ref[...]

    k = jnp.dot(wkv_ref[0:DIM, :], xs, preferred_element_type=f32)

    m = jnp.maximum(k, _rot(k, 1))
    m = jnp.maximum(m, _rot(m, 2))
    m = jnp.maximum(m, _rot(m, 32))
    m = jnp.maximum(m, _rot(m, 64))
    a_field = jnp.dot(q, m, preferred_element_type=f32)

    l_i = jax.lax.broadcasted_iota(jnp.int32, (NREG, NLOC), 1)
    r_i = jax.lax.broadcasted_iota(jnp.int32, (NREG, NLOC), 0)
    rmap = (((l_i // 128) * 8 + (l_i % 32) // 4) == r_i).astype(f32)
    is_rep = ((l_i % 4) == 0) & (((l_i // 32) % 4) == 0)
    emat = jnp.where(is_rep, rmap, 0.0)
    a = jax.lax.dot_general(a_field, emat, _NT,
                            preferred_element_type=f32)

    arow = jnp.broadcast_to(a, (NREG, NREG))
    acol = arow.T
    ii = jax.lax.broadcasted_iota(jnp.int32, (NREG, NREG), 0)
    jj = jax.lax.broadcasted_iota(jnp.int32, (NREG, NREG), 1)
    beats = (acol > arow) | ((acol == arow) & (ii < jj))
    rank = jnp.sum(beats.astype(f32), axis=0, keepdims=True)
    rankloc = jnp.dot(rank, rmap, preferred_element_type=f32)
    maskloc = rankloc < float(TOPK)

    hidx = jax.lax.broadcasted_iota(jnp.int32, (NH, DIM), 0)
    cidx = jax.lax.broadcasted_iota(jnp.int32, (NH, DIM), 1)
    diag = (cidx // HD) == hidx
    qm = jnp.where(diag, jnp.broadcast_to(q, (NH, DIM)), 0.0)
    s = jnp.dot(qm, k, preferred_element_type=f32) * SCALE
    s = jnp.where(maskloc, s, -1e30)
    s = s - jnp.max(s, axis=1, keepdims=True)
    e = jnp.exp(s)
    attn = e / jnp.sum(e, axis=1, keepdims=True)

    z = jax.lax.dot_general(attn, xs, _NT, preferred_element_type=f32)

    o_full = jax.lax.dot_general(z, wkv_ref[DIM:, :], _NT,
                                 preferred_element_type=f32)
    o_vec = jnp.sum(jnp.where(diag, o_full, 0.0), axis=0,
                    keepdims=True) + bv_ref[...]

    out_ref[0] = (jax.lax.dot_general(o_vec, wo_ref[...], _NT,
                                      preferred_element_type=f32)
                  + bo_ref[...])


def kernel(x, Wq, bq, Wkv, bkv, Wo, bo):
    xs = x[:, 1:].reshape(BATCH, DIM, NLOC)
    x0 = x[:, 0].reshape(BATCH, 1, DIM)
    bq2 = bq.reshape(1, DIM)
    bv2 = bkv[DIM:].reshape(1, DIM)
    bo2 = bo.reshape(1, OUT_DIM)

    out = pl.pallas_call(
        _bra_kernel,
        grid=(BATCH,),
        in_specs=[
            pl.BlockSpec((1, DIM, NLOC), lambda b: (b, 0, 0)),
            pl.BlockSpec((1, 1, DIM), lambda b: (b, 0, 0)),
            pl.BlockSpec((DIM, DIM), lambda b: (0, 0)),
            pl.BlockSpec((1, DIM), lambda b: (0, 0)),
            pl.BlockSpec((OUT_DIM, DIM), lambda b: (0, 0)),
            pl.BlockSpec((1, DIM), lambda b: (0, 0)),
            pl.BlockSpec((OUT_DIM, DIM), lambda b: (0, 0)),
            pl.BlockSpec((1, OUT_DIM), lambda b: (0, 0)),
        ],
        out_specs=pl.BlockSpec((1, 1, OUT_DIM), lambda b: (b, 0, 0)),
        out_shape=jax.ShapeDtypeStruct((BATCH, 1, OUT_DIM), jnp.float32),
    )(xs, x0, Wq, bq2, Wkv, bv2, Wo, bo2)
    return out.reshape(BATCH, OUT_DIM)
